# baseline (speedup 1.0000x reference)
import functools

import jax
import jax.numpy as jnp
from jax import lax
from jax.experimental import pallas as pl
from jax.experimental.pallas import tpu as pltpu
from jax.experimental.pallas import tpu_sc as plsc

_B = 16384
_D = 32
_NU = 1000000
_NI = 100000
_NC = 2
_NS = 16
_NW = _NC * _NS
_BPW = _B // _NW
_CH = 128
_NQ = _BPW // _CH

_RC = 64
_OU = _NU // 4
_OI = _NI // 4

_compiler_params = pltpu.CompilerParams(needs_layout_passes=False,
                                        use_tc_tiling_on_sc=True)
_mesh = plsc.VectorSubcoreMesh(core_axis_name="c", subcore_axis_name="s")


def _repack_body(u_emb, i_emb, u2, i2,
                 buf0, buf1, ob0, ob1, sin0, sin1, sout0, sout1):
    wid = lax.axis_index("s") * _NC + lax.axis_index("c")
    bufs = (buf0, buf1)
    obufs = (ob0, ob1)
    sins = (sin0, sin1)
    souts = (sout0, sout1)

    def run(tbl, out, total_or):
        per_w = (-(-total_or // _NW) + 7) // 8 * 8
        n_chunks = -(-per_w // _RC)
        start = wid * per_w
        n_or = jnp.minimum(per_w, total_or - start)

        def orow(c):
            return pl.multiple_of(start + jnp.minimum(c * _RC, n_or - _RC), 8)

        def fire(c):
            return pltpu.async_copy(tbl.at[pl.ds(orow(c) * 4, _RC * 4), :],
                                    bufs[c % 2], sins[c % 2])

        pend_in = fire(0)
        pend_out = [None, None]
        for c in range(n_chunks):
            nxt = fire(c + 1) if c + 1 < n_chunks else None
            pend_in.wait()
            pend_in = nxt
            if pend_out[c % 2] is not None:
                pend_out[c % 2].wait()
            buf = bufs[c % 2]
            ob = obufs[c % 2]

            def row(t, carry):
                for j in range(8):
                    k = t * 128 + j * 16
                    ob[t, pl.ds(j * 16, 16)] = buf[k // 32, pl.ds(k % 32, 16)]
                return carry

            lax.fori_loop(0, _RC, row, 0)
            pend_out[c % 2] = pltpu.async_copy(
                ob, out.at[pl.ds(orow(c), _RC), :], souts[c % 2])
        for p in pend_out:
            if p is not None:
                p.wait()

    run(u_emb, u2, _OU)
    run(i_emb, i2, _OI)


_repack = functools.partial(
    pl.kernel,
    mesh=_mesh,
    compiler_params=_compiler_params,
    out_type=(jax.ShapeDtypeStruct((_OU, 128), jnp.float32),
              jax.ShapeDtypeStruct((_OI, 128), jnp.float32)),
    scratch_types=[
        pltpu.VMEM((_RC * 4, _D), jnp.float32),
        pltpu.VMEM((_RC * 4, _D), jnp.float32),
        pltpu.VMEM((_RC, 128), jnp.float32),
        pltpu.VMEM((_RC, 128), jnp.float32),
        pltpu.SemaphoreType.DMA,
        pltpu.SemaphoreType.DMA,
        pltpu.SemaphoreType.DMA,
        pltpu.SemaphoreType.DMA,
    ],
)(_repack_body)


def _sc_body(users, items, u_bias1, i_bias1, u_emb2, i_emb2, out,
             uidx, iidx, uq, iq, ub, ib,
             urb0, urb1, irb0, irb1, outv,
             semb, semu0, semu1, semi0, semi1):
    wid = lax.axis_index("s") * _NC + lax.axis_index("c")
    base = wid * _BPW

    pltpu.sync_copy(users.at[pl.ds(base, _BPW)], uidx)
    pltpu.sync_copy(items.at[pl.ds(base, _BPW)], iidx)

    for k in range(_BPW // 16):
        s = pl.ds(k * 16, 16)
        uq[s] = lax.shift_right_logical(uidx[s], 2)
        iq[s] = lax.shift_right_logical(iidx[s], 2)

    bcopies = []
    for j in range(_NQ):
        s = pl.ds(j * _CH, _CH)
        bcopies.append(pltpu.async_copy(u_bias1.at[uidx.at[s]], ub.at[s], semb))
        bcopies.append(pltpu.async_copy(i_bias1.at[iidx.at[s]], ib.at[s], semb))

    ubufs = (urb0, urb1)
    ibufs = (irb0, irb1)
    usems = (semu0, semu1)
    isems = (semi0, semi1)

    def fire(q):
        s = pl.ds(q * _CH, _CH)
        return (
            pltpu.async_copy(u_emb2.at[uq.at[s]], ubufs[q % 2], usems[q % 2]),
            pltpu.async_copy(i_emb2.at[iq.at[s]], ibufs[q % 2], isems[q % 2]),
        )

    lane = lax.iota(jnp.int32, 16)
    pending = fire(0)
    for q in range(_NQ):
        nxt = fire(q + 1) if q + 1 < _NQ else None
        for c in pending:
            c.wait()
        pending = nxt
        bufu = ubufs[q % 2]
        bufi = ibufs[q % 2]

        def group(g, carry):
            gb = q * _CH + g * 16
            s = pl.ds(gb, 16)
            usub = (uidx[s] & 3) * _D
            isub = (iidx[s] & 3) * _D
            rowv = g * 16 + lane
            acc = jnp.zeros((16,), jnp.float32)
            for d in range(_D):
                uv = plsc.load_gather(bufu, [rowv, usub + d])
                iv = plsc.load_gather(bufi, [rowv, isub + d])
                acc = acc + uv * iv
            outv[s] = acc
            return carry

        lax.fori_loop(0, _CH // 16, group, 0)

    for c in bcopies:
        c.wait()
    for k in range(_BPW // 16):
        s = pl.ds(k * 16, 16)
        outv[s] = outv[s] + ub[s] + ib[s]

    pltpu.sync_copy(outv, out.at[pl.ds(base, _BPW)])


_score = functools.partial(
    pl.kernel,
    mesh=_mesh,
    compiler_params=_compiler_params,
    out_type=jax.ShapeDtypeStruct((_B,), jnp.float32),
    scratch_types=[
        pltpu.VMEM((_BPW,), jnp.int32),
        pltpu.VMEM((_BPW,), jnp.int32),
        pltpu.VMEM((_BPW,), jnp.int32),
        pltpu.VMEM((_BPW,), jnp.int32),
        pltpu.VMEM((_BPW,), jnp.float32),
        pltpu.VMEM((_BPW,), jnp.float32),
        pltpu.VMEM((_CH, 128), jnp.float32),
        pltpu.VMEM((_CH, 128), jnp.float32),
        pltpu.VMEM((_CH, 128), jnp.float32),
        pltpu.VMEM((_CH, 128), jnp.float32),
        pltpu.VMEM((_BPW,), jnp.float32),
        pltpu.SemaphoreType.DMA,
        pltpu.SemaphoreType.DMA,
        pltpu.SemaphoreType.DMA,
        pltpu.SemaphoreType.DMA,
        pltpu.SemaphoreType.DMA,
    ],
)(_sc_body)


@jax.jit
def kernel(users, items, u_bias_w, i_bias_w, u_embed_w, i_embed_w):
    u2, i2 = _repack(u_embed_w, i_embed_w)
    return _score(users.astype(jnp.int32), items.astype(jnp.int32),
                  u_bias_w[:, 0], i_bias_w[:, 0], u2, i2)


# speedup vs baseline: 1.1701x; 1.1701x over previous
import functools

import jax
import jax.numpy as jnp
from jax import lax
from jax.experimental import pallas as pl
from jax.experimental.pallas import tpu as pltpu
from jax.experimental.pallas import tpu_sc as plsc

_B = 16384
_D = 32
_NU = 1000000
_NI = 100000
_NC = 2
_NS = 16
_NW = _NC * _NS
_BPW = _B // _NW
_CH = 128
_NQ = _BPW // _CH


def _sc_body(users, items, u_bias1, i_bias1, u_emb2, i_emb2, out,
             uidx, iidx, ub, ib,
             urb0, urb1, irb0, irb1, outv,
             semb, semu0, semu1, semi0, semi1):
    wid = lax.axis_index("s") * _NC + lax.axis_index("c")
    base = wid * _BPW

    pltpu.sync_copy(users.at[pl.ds(base, _BPW)], uidx)
    pltpu.sync_copy(items.at[pl.ds(base, _BPW)], iidx)

    bcopies = []
    for j in range(_NQ):
        s = pl.ds(j * _CH, _CH)
        bcopies.append(pltpu.async_copy(u_bias1.at[uidx.at[s]], ub.at[s], semb))
        bcopies.append(pltpu.async_copy(i_bias1.at[iidx.at[s]], ib.at[s], semb))

    ubufs = (urb0, urb1)
    ibufs = (irb0, irb1)
    usems = (semu0, semu1)
    isems = (semi0, semi1)

    def fire(q):
        s = pl.ds(q * _CH, _CH)
        return (
            pltpu.async_copy(u_emb2.at[uidx.at[s]], ubufs[q % 2], usems[q % 2]),
            pltpu.async_copy(i_emb2.at[iidx.at[s]], ibufs[q % 2], isems[q % 2]),
        )

    lane = lax.iota(jnp.int32, 16)
    dvec = jnp.zeros((16,), jnp.int32)
    pending = fire(0)
    for q in range(_NQ):
        nxt = fire(q + 1) if q + 1 < _NQ else None
        for c in pending:
            c.wait()
        pending = nxt
        bufu = ubufs[q % 2]
        bufi = ibufs[q % 2]

        def group(g, carry):
            gb = q * _CH + g * 16
            s = pl.ds(gb, 16)
            rowv = g * 16 + lane
            acc = jnp.zeros((16,), jnp.float32)
            for d in range(_D):
                uv = plsc.load_gather(bufu, [rowv, dvec + d])
                iv = plsc.load_gather(bufi, [rowv, dvec + d])
                acc = acc + uv * iv
            outv[s] = acc
            return carry

        lax.fori_loop(0, _CH // 16, group, 0)

    for c in bcopies:
        c.wait()
    for k in range(_BPW // 16):
        s = pl.ds(k * 16, 16)
        outv[s] = outv[s] + ub[s] + ib[s]

    pltpu.sync_copy(outv, out.at[pl.ds(base, _BPW)])


_mesh = plsc.VectorSubcoreMesh(core_axis_name="c", subcore_axis_name="s")

_score = functools.partial(
    pl.kernel,
    mesh=_mesh,
    compiler_params=pltpu.CompilerParams(needs_layout_passes=False,
                                         use_tc_tiling_on_sc=True),
    out_type=jax.ShapeDtypeStruct((_B,), jnp.float32),
    scratch_types=[
        pltpu.VMEM((_BPW,), jnp.int32),
        pltpu.VMEM((_BPW,), jnp.int32),
        pltpu.VMEM((_BPW,), jnp.float32),
        pltpu.VMEM((_BPW,), jnp.float32),
        pltpu.VMEM((_CH, 128), jnp.float32),
        pltpu.VMEM((_CH, 128), jnp.float32),
        pltpu.VMEM((_CH, 128), jnp.float32),
        pltpu.VMEM((_CH, 128), jnp.float32),
        pltpu.VMEM((_BPW,), jnp.float32),
        pltpu.SemaphoreType.DMA,
        pltpu.SemaphoreType.DMA,
        pltpu.SemaphoreType.DMA,
        pltpu.SemaphoreType.DMA,
        pltpu.SemaphoreType.DMA,
    ],
)(_sc_body)


@jax.jit
def kernel(users, items, u_bias_w, i_bias_w, u_embed_w, i_embed_w):
    return _score(users.astype(jnp.int32), items.astype(jnp.int32),
                  u_bias_w[:, 0], i_bias_w[:, 0],
                  jnp.pad(u_embed_w, ((0, 0), (0, 128 - _D))),
                  jnp.pad(i_embed_w, ((0, 0), (0, 128 - _D))))
